# baseline (speedup 1.0000x reference)
import jax
import jax.numpy as jnp
from jax.experimental import pallas as pl
from jax.experimental.pallas import tpu as pltpu

_G = 3


def _tc_mean(x, block_rows):
    g, rows, d = x.shape

    def body(x_ref, o_ref):
        o_ref[...] = (x_ref[0] + x_ref[1] + x_ref[2]) * (1.0 / _G)

    return pl.pallas_call(
        body,
        grid=(rows // block_rows,),
        in_specs=[pl.BlockSpec((g, block_rows, d), lambda i: (0, i, 0))],
        out_specs=pl.BlockSpec((block_rows, d), lambda i: (i, 0)),
        out_shape=jax.ShapeDtypeStruct((rows, d), jnp.float32),
        compiler_params=pltpu.CompilerParams(
            dimension_semantics=("parallel",),
        ),
    )(x)


def kernel(time_emb, expert_embeddings, time_step, total_steps):
    del time_emb, time_step, total_steps
    G, B, L, D = expert_embeddings.shape
    rows = B * L
    x = expert_embeddings.reshape(G, rows, D)
    out = _tc_mean(x, 512)
    return out.reshape(B, L, D)


# speedup vs baseline: 1.0016x; 1.0016x over previous
import jax
import jax.numpy as jnp
from jax.experimental import pallas as pl
from jax.experimental.pallas import tpu as pltpu

_G = 3


def _tc_mean(x, block_rows):
    g, rows, d = x.shape

    def body(a_ref, b_ref, c_ref, o_ref):
        o_ref[...] = (a_ref[0] + b_ref[0] + c_ref[0]) * (1.0 / _G)

    return pl.pallas_call(
        body,
        grid=(rows // block_rows,),
        in_specs=[
            pl.BlockSpec((1, block_rows, d), lambda i, g=gi: (g, i, 0))
            for gi in range(g)
        ],
        out_specs=pl.BlockSpec((block_rows, d), lambda i: (i, 0)),
        out_shape=jax.ShapeDtypeStruct((rows, d), jnp.float32),
        compiler_params=pltpu.CompilerParams(
            dimension_semantics=("parallel",),
        ),
    )(x, x, x)


def kernel(time_emb, expert_embeddings, time_step, total_steps):
    del time_emb, time_step, total_steps
    G, B, L, D = expert_embeddings.shape
    rows = B * L
    x = expert_embeddings.reshape(G, rows, D)
    out = _tc_mean(x, 512)
    return out.reshape(B, L, D)
